# baseline (speedup 1.0000x reference)
import functools

import jax
import jax.numpy as jnp
from jax import lax
from jax.experimental import pallas as pl
from jax.experimental.pallas import tpu as pltpu
from jax.experimental.pallas import tpu_sc as plsc

B = 16384
D = 64
L = 16

_info = plsc.get_sparse_core_info()
NC, NS = _info.num_cores, _info.num_subcores
NW = NC * NS
BPW = B // NW

_mesh = plsc.VectorSubcoreMesh(core_axis_name="c", subcore_axis_name="s")


@functools.partial(
    pl.kernel,
    mesh=_mesh,
    compiler_params=pltpu.CompilerParams(
        needs_layout_passes=False, use_tc_tiling_on_sc=False),
    out_type=jax.ShapeDtypeStruct((B,), jnp.float32),
    scratch_types=[
        pltpu.VMEM((BPW,), jnp.int32),
        pltpu.VMEM((BPW,), jnp.int32),
        pltpu.VMEM((BPW, D), jnp.float32),
        pltpu.VMEM((BPW, D), jnp.float32),
        pltpu.VMEM((BPW,), jnp.float32),
        pltpu.VMEM((BPW,), jnp.float32),
        pltpu.VMEM((BPW,), jnp.float32),
        pltpu.SemaphoreType.DMA,
    ],
)
def _cf_kernel(uid_hbm, mid_hbm, uemb_hbm, memb_hbm, ub_hbm, mb_hbm,
               out_hbm, uid_v, mid_v, urows, mrows, ub_v, mb_v, out_v, sem):
    wid = lax.axis_index("s") * NC + lax.axis_index("c")
    base = wid * BPW

    pltpu.sync_copy(uid_hbm.at[pl.ds(base, BPW)], uid_v)
    pltpu.sync_copy(mid_hbm.at[pl.ds(base, BPW)], mid_v)

    cp1 = pltpu.async_copy(uemb_hbm.at[uid_v], urows, sem)
    cp2 = pltpu.async_copy(memb_hbm.at[mid_v], mrows, sem)
    cp3 = pltpu.async_copy(ub_hbm.at[uid_v], ub_v, sem)
    cp4 = pltpu.async_copy(mb_hbm.at[mid_v], mb_v, sem)
    cp1.wait()
    cp2.wait()
    cp3.wait()
    cp4.wait()

    def init_body(g, _):
        b0 = g * L
        out_v[pl.ds(b0, L)] = ub_v[pl.ds(b0, L)] + mb_v[pl.ds(b0, L)]
        return 0

    lax.fori_loop(0, BPW // L, init_body, 0)

    def body(b, _):
        p = urows[b, pl.ds(0, L)] * mrows[b, pl.ds(0, L)]
        p = p + urows[b, pl.ds(L, L)] * mrows[b, pl.ds(L, L)]
        p = p + urows[b, pl.ds(2 * L, L)] * mrows[b, pl.ds(2 * L, L)]
        p = p + urows[b, pl.ds(3 * L, L)] * mrows[b, pl.ds(3 * L, L)]
        plsc.addupdate_scatter(out_v, [jnp.full((L,), b, jnp.int32)], p)
        return 0

    lax.fori_loop(0, BPW, body, 0, unroll=4)

    pltpu.sync_copy(out_v, out_hbm.at[pl.ds(base, BPW)])


def kernel(user_ids, movie_ids, user_emb_table, movie_emb_table,
           user_bias_table, movie_bias_table):
    return _cf_kernel(
        user_ids.astype(jnp.int32),
        movie_ids.astype(jnp.int32),
        user_emb_table,
        movie_emb_table,
        user_bias_table.reshape(-1),
        movie_bias_table.reshape(-1),
    )
